# baseline (speedup 1.0000x reference)
import functools
import jax
import jax.numpy as jnp
from jax import lax
from jax.experimental import pallas as pl
from jax.experimental.pallas import tpu as pltpu
from jax.experimental.pallas import tpu_sc as plsc

_LANES = 16
_CHUNK_ROWS = 32
_N_WORKERS = 32


def _make_sc_add(total_rows, seq_len, d_model):
    rows_per_w = total_rows // _N_WORKERS
    n_chunks = rows_per_w // _CHUNK_ROWS
    chunk_elems = _CHUNK_ROWS * d_model

    mesh = plsc.VectorSubcoreMesh(core_axis_name="c", subcore_axis_name="s")

    @functools.partial(
        pl.kernel,
        mesh=mesh,
        out_type=jax.ShapeDtypeStruct((total_rows * d_model,), jnp.float32),
        scratch_types=[
            pltpu.VMEM((chunk_elems,), jnp.float32),
            pltpu.VMEM((chunk_elems,), jnp.float32),
        ],
    )
    def k(x_hbm, t_hbm, o_hbm, xbuf, tbuf):
        wid = lax.axis_index("s") * 2 + lax.axis_index("c")
        row0 = wid * rows_per_w

        def chunk(ci, carry):
            r0 = row0 + ci * _CHUNK_ROWS
            s0 = lax.rem(r0, seq_len)
            pltpu.sync_copy(x_hbm.at[pl.ds(r0 * d_model, chunk_elems)], xbuf)
            pltpu.sync_copy(t_hbm.at[pl.ds(s0 * d_model, chunk_elems)], tbuf)

            def add16(i, c2):
                off = i * _LANES
                xbuf[pl.ds(off, _LANES)] = (
                    xbuf[pl.ds(off, _LANES)] + tbuf[pl.ds(off, _LANES)]
                )
                return c2

            lax.fori_loop(0, chunk_elems // _LANES, add16, 0)
            pltpu.sync_copy(xbuf, o_hbm.at[pl.ds(r0 * d_model, chunk_elems)])
            return carry

        lax.fori_loop(0, n_chunks, chunk, 0)

    return k


def kernel(x, pos_table):
    batch, seq_len, d_model = x.shape
    total_rows = batch * seq_len
    x_flat = x.reshape(total_rows * d_model)
    t_flat = pos_table[:seq_len].reshape(seq_len * d_model)
    k = _make_sc_add(total_rows, seq_len, d_model)
    out = k(x_flat, t_flat)
    return out.reshape(batch, seq_len, d_model)


# speedup vs baseline: 1.4566x; 1.4566x over previous
import functools
import jax
import jax.numpy as jnp
from jax import lax
from jax.experimental import pallas as pl
from jax.experimental.pallas import tpu as pltpu
from jax.experimental.pallas import tpu_sc as plsc

_LANES = 16
_CHUNK_ROWS = 16
_NBUF = 2
_UNROLL = 8
_N_WORKERS = 32


def _make_sc_add(total_rows, seq_len, d_model):
    rows_per_w = total_rows // _N_WORKERS
    n_chunks = rows_per_w // _CHUNK_ROWS
    n_groups = n_chunks // _NBUF
    chunk_elems = _CHUNK_ROWS * d_model

    mesh = plsc.VectorSubcoreMesh(core_axis_name="c", subcore_axis_name="s")

    @functools.partial(
        pl.kernel,
        mesh=mesh,
        out_type=jax.ShapeDtypeStruct((total_rows * d_model,), jnp.float32),
        scratch_types=[
            pltpu.VMEM((_NBUF, chunk_elems), jnp.float32),
            pltpu.VMEM((_NBUF, chunk_elems), jnp.float32),
            pltpu.SemaphoreType.DMA((_NBUF,)),
            pltpu.SemaphoreType.DMA((_NBUF,)),
        ],
    )
    def k(x_hbm, t_hbm, o_hbm, xbuf, tbuf, sx, st):
        wid = lax.axis_index("s") * 2 + lax.axis_index("c")
        row0 = wid * rows_per_w

        def start_in(ci, b):
            base = row0 + ci * _CHUNK_ROWS
            pltpu.make_async_copy(
                x_hbm.at[pl.ds(base * d_model, chunk_elems)], xbuf.at[b], sx.at[b]
            ).start()
            pltpu.make_async_copy(
                t_hbm.at[pl.ds(lax.rem(base, seq_len) * d_model, chunk_elems)],
                tbuf.at[b],
                st.at[b],
            ).start()

        for b in range(_NBUF):
            start_in(b, b)

        def group(g, carry):
            for b in range(_NBUF):
                ci = g * _NBUF + b
                pltpu.make_async_copy(
                    x_hbm.at[pl.ds(0, chunk_elems)], xbuf.at[b], sx.at[b]
                ).wait()
                pltpu.make_async_copy(
                    t_hbm.at[pl.ds(0, chunk_elems)], tbuf.at[b], st.at[b]
                ).wait()

                def add_u(i, c2, _b=b):
                    off = i * (_LANES * _UNROLL)
                    for u in range(_UNROLL):
                        o2 = off + u * _LANES
                        xbuf[_b, pl.ds(o2, _LANES)] = (
                            xbuf[_b, pl.ds(o2, _LANES)] + tbuf[_b, pl.ds(o2, _LANES)]
                        )
                    return c2

                lax.fori_loop(0, chunk_elems // (_LANES * _UNROLL), add_u, 0)

                base = row0 + ci * _CHUNK_ROWS
                pltpu.sync_copy(
                    xbuf.at[b], o_hbm.at[pl.ds(base * d_model, chunk_elems)]
                )

                @pl.when(ci + _NBUF < n_chunks)
                def _():
                    start_in(ci + _NBUF, b)

            return carry

        lax.fori_loop(0, n_groups, group, 0)

    return k


def kernel(x, pos_table):
    batch, seq_len, d_model = x.shape
    total_rows = batch * seq_len
    x_flat = x.reshape(total_rows * d_model)
    t_flat = pos_table[:seq_len].reshape(seq_len * d_model)
    k = _make_sc_add(total_rows, seq_len, d_model)
    out = k(x_flat, t_flat)
    return out.reshape(batch, seq_len, d_model)


# speedup vs baseline: 1.7772x; 1.2201x over previous
import functools
import jax
import jax.numpy as jnp
from jax import lax
from jax.experimental import pallas as pl
from jax.experimental.pallas import tpu as pltpu
from jax.experimental.pallas import tpu_sc as plsc

_LANES = 16
_CHUNK_ROWS = 16
_NBUF = 2
_UNROLL = 8
_N_WORKERS = 32


def _make_sc_add(batch, seq_len, d_model):
    total_rows = batch * seq_len
    rows_per_w = total_rows // _N_WORKERS
    n_chunks = rows_per_w // _CHUNK_ROWS
    n_groups = n_chunks // _NBUF
    w_per_batch = _N_WORKERS // batch
    slices_per_row = d_model // _LANES

    mesh = plsc.VectorSubcoreMesh(core_axis_name="c", subcore_axis_name="s")

    @functools.partial(
        pl.kernel,
        mesh=mesh,
        out_type=jax.ShapeDtypeStruct((batch, seq_len, d_model), jnp.float32),
        scratch_types=[
            pltpu.VMEM((_NBUF, _CHUNK_ROWS, d_model), jnp.float32),
            pltpu.VMEM((_NBUF, _CHUNK_ROWS, d_model), jnp.float32),
            pltpu.VMEM((_NBUF, _CHUNK_ROWS, d_model), jnp.float32),
            pltpu.SemaphoreType.DMA((_NBUF,)),
            pltpu.SemaphoreType.DMA((_NBUF,)),
            pltpu.SemaphoreType.DMA((_NBUF,)),
        ],
    )
    def k(x_hbm, t_hbm, o_hbm, xbuf, tbuf, obuf, sx, st, so):
        wid = lax.axis_index("s") * 2 + lax.axis_index("c")
        b_idx = wid // w_per_batch
        s_base = (wid % w_per_batch) * rows_per_w

        def start_in(ci, slot):
            s0 = s_base + ci * _CHUNK_ROWS
            pltpu.make_async_copy(
                x_hbm.at[b_idx, pl.ds(s0, _CHUNK_ROWS), :], xbuf.at[slot], sx.at[slot]
            ).start()
            pltpu.make_async_copy(
                t_hbm.at[pl.ds(s0, _CHUNK_ROWS), :], tbuf.at[slot], st.at[slot]
            ).start()

        for b in range(_NBUF):
            start_in(b, b)

        def group(g, carry):
            for b in range(_NBUF):
                ci = g * _NBUF + b
                pltpu.make_async_copy(
                    x_hbm.at[0, pl.ds(0, _CHUNK_ROWS), :], xbuf.at[b], sx.at[b]
                ).wait()
                pltpu.make_async_copy(
                    t_hbm.at[pl.ds(0, _CHUNK_ROWS), :], tbuf.at[b], st.at[b]
                ).wait()

                def row_add(r, c2, _b=b):
                    def grp_add(gg, c3, _r=r, _b2=_b):
                        for u in range(_UNROLL):
                            c0 = (gg * _UNROLL + u) * _LANES
                            obuf[_b2, _r, pl.ds(c0, _LANES)] = (
                                xbuf[_b2, _r, pl.ds(c0, _LANES)]
                                + tbuf[_b2, _r, pl.ds(c0, _LANES)]
                            )
                        return c3

                    lax.fori_loop(0, slices_per_row // _UNROLL, grp_add, 0)
                    return c2

                lax.fori_loop(0, _CHUNK_ROWS, row_add, 0)

                @pl.when(ci + _NBUF < n_chunks)
                def _():
                    start_in(ci + _NBUF, b)

                @pl.when(ci >= _NBUF)
                def _():
                    pltpu.make_async_copy(
                        o_hbm.at[0, pl.ds(0, _CHUNK_ROWS), :], obuf.at[b], so.at[b]
                    ).wait()

                s0 = s_base + ci * _CHUNK_ROWS
                pltpu.make_async_copy(
                    obuf.at[b], o_hbm.at[b_idx, pl.ds(s0, _CHUNK_ROWS), :], so.at[b]
                ).start()

            return carry

        lax.fori_loop(0, n_groups, group, 0)

        for b in range(_NBUF):
            pltpu.make_async_copy(
                o_hbm.at[0, pl.ds(0, _CHUNK_ROWS), :], obuf.at[b], so.at[b]
            ).wait()

    return k


def kernel(x, pos_table):
    batch, seq_len, d_model = x.shape
    table = pos_table[:seq_len]
    k = _make_sc_add(batch, seq_len, d_model)
    return k(x, table)


# speedup vs baseline: 2.1951x; 1.2351x over previous
import functools
import jax
import jax.numpy as jnp
from jax import lax
from jax.experimental import pallas as pl
from jax.experimental.pallas import tpu as pltpu
from jax.experimental.pallas import tpu_sc as plsc

_LANES = 16
_CHUNK_ROWS = 16
_NBUF = 2
_UNROLL = 8
_N_WORKERS = 32


def _make_sc_add(batch, seq_len, d_model):
    total_rows = batch * seq_len
    rows_per_w = total_rows // _N_WORKERS
    n_chunks = rows_per_w // _CHUNK_ROWS
    n_groups = n_chunks // _NBUF
    w_per_batch = _N_WORKERS // batch
    slices_per_row = d_model // _LANES

    mesh = plsc.VectorSubcoreMesh(core_axis_name="c", subcore_axis_name="s")

    @functools.partial(
        pl.kernel,
        mesh=mesh,
        out_type=jax.ShapeDtypeStruct((batch, seq_len, d_model), jnp.float32),
        scratch_types=[
            pltpu.VMEM((_NBUF, _CHUNK_ROWS, d_model), jnp.float32),
            pltpu.VMEM((_NBUF, _CHUNK_ROWS, d_model), jnp.float32),
            pltpu.VMEM((_NBUF, _CHUNK_ROWS, d_model), jnp.float32),
            pltpu.SemaphoreType.DMA((_NBUF,)),
            pltpu.SemaphoreType.DMA((_NBUF,)),
            pltpu.SemaphoreType.DMA((_NBUF,)),
        ],
    )
    def k(x_hbm, t_hbm, o_hbm, xbuf, tbuf, obuf, sx, st, so):
        wid = lax.axis_index("s") * 2 + lax.axis_index("c")
        b_idx = wid // w_per_batch
        s_base = (wid % w_per_batch) * rows_per_w

        def start_in(ci, slot):
            s0 = s_base + ci * _CHUNK_ROWS
            pltpu.make_async_copy(
                x_hbm.at[b_idx, pl.ds(s0, _CHUNK_ROWS), :], xbuf.at[slot], sx.at[slot]
            ).start()
            pltpu.make_async_copy(
                t_hbm.at[pl.ds(s0, _CHUNK_ROWS), :], tbuf.at[slot], st.at[slot]
            ).start()

        for b in range(_NBUF):
            start_in(b, b)

        def group(g, carry):
            for b in range(_NBUF):
                ci = g * _NBUF + b
                pltpu.make_async_copy(
                    x_hbm.at[0, pl.ds(0, _CHUNK_ROWS), :], xbuf.at[b], sx.at[b]
                ).wait()
                pltpu.make_async_copy(
                    t_hbm.at[pl.ds(0, _CHUNK_ROWS), :], tbuf.at[b], st.at[b]
                ).wait()

                for r in range(_CHUNK_ROWS):

                    def grp_add(gg, c3, _b2=b, _r=r):
                        for u in range(_UNROLL):
                            c0 = (gg * _UNROLL + u) * _LANES
                            obuf[_b2, _r, pl.ds(c0, _LANES)] = (
                                xbuf[_b2, _r, pl.ds(c0, _LANES)]
                                + tbuf[_b2, _r, pl.ds(c0, _LANES)]
                            )
                        return c3

                    lax.fori_loop(0, slices_per_row // _UNROLL, grp_add, 0)

                @pl.when(ci + _NBUF < n_chunks)
                def _():
                    start_in(ci + _NBUF, b)

                @pl.when(ci >= _NBUF)
                def _():
                    pltpu.make_async_copy(
                        o_hbm.at[0, pl.ds(0, _CHUNK_ROWS), :], obuf.at[b], so.at[b]
                    ).wait()

                s0 = s_base + ci * _CHUNK_ROWS
                pltpu.make_async_copy(
                    obuf.at[b], o_hbm.at[b_idx, pl.ds(s0, _CHUNK_ROWS), :], so.at[b]
                ).start()

            return carry

        lax.fori_loop(0, n_groups, group, 0)

        for b in range(_NBUF):
            pltpu.make_async_copy(
                o_hbm.at[0, pl.ds(0, _CHUNK_ROWS), :], obuf.at[b], so.at[b]
            ).wait()

    return k


def kernel(x, pos_table):
    batch, seq_len, d_model = x.shape
    table = pos_table[:seq_len]
    k = _make_sc_add(batch, seq_len, d_model)
    return k(x, table)
